# baseline (speedup 1.0000x reference)
import functools

import jax
import jax.numpy as jnp
from jax import lax
from jax.experimental import pallas as pl
from jax.experimental.pallas import tpu as pltpu
from jax.experimental.pallas import tpu_sc as plsc

_info = plsc.get_sparse_core_info()
_NC, _NS = _info.num_cores, _info.num_subcores
_NW = _NC * _NS


def _make_router(n_tokens, vocab):
    assert n_tokens % (8 * _NW) == 0
    per_w = n_tokens // _NW
    mesh = plsc.VectorSubcoreMesh(core_axis_name="c", subcore_axis_name="s")

    @functools.partial(
        pl.kernel,
        mesh=mesh,
        out_type=jax.ShapeDtypeStruct((n_tokens,), jnp.int32),
        scratch_types=[
            pltpu.VMEM((per_w,), jnp.int32),
            pltpu.VMEM((per_w,), jnp.int32),
            pltpu.VMEM_SHARED((vocab,), jnp.int32),
            pltpu.SemaphoreType.DMA,
            pltpu.SemaphoreType.DMA,
        ],
    )
    def router(ids_hbm, table_hbm, out_hbm, idx_v, vals_v, table_s, s_in, s_g):
        sid = lax.axis_index("s")
        wid = sid * _NC + lax.axis_index("c")
        base = wid * per_w
        in_c = pltpu.async_copy(ids_hbm.at[pl.ds(base, per_w)], idx_v, s_in)

        @pl.when(sid == 0)
        def _stage_table():
            pltpu.sync_copy(table_hbm, table_s)

        plsc.subcore_barrier()
        in_c.wait()
        pltpu.async_copy(table_s.at[idx_v], vals_v, s_g).wait()
        pltpu.sync_copy(vals_v, out_hbm.at[pl.ds(base, per_w)])

    return router


def kernel(input, hash):
    b, s = input.shape
    n = b * s
    ids = input.astype(jnp.int32).reshape(n)
    out = _make_router(n, hash.shape[0])(ids, hash.astype(jnp.int32))
    return out.reshape(b, s).astype(hash.dtype)


# speedup vs baseline: 1.0252x; 1.0252x over previous
import jax
import jax.numpy as jnp
from jax import lax
from jax.experimental import pallas as pl
from jax.experimental.pallas import tpu as pltpu
from jax.experimental.pallas import tpu_sc as plsc
from jax._src.pallas import mpmd

_info = plsc.get_sparse_core_info()
_NC, _NS = _info.num_cores, _info.num_subcores
_NW = _NC * _NS


def _make_router(n_tokens, vocab):
    assert n_tokens % (8 * _NW) == 0
    per_w = n_tokens // _NW
    smesh = plsc.ScalarSubcoreMesh(axis_name="c", num_cores=_NC)
    vmesh = plsc.VectorSubcoreMesh(core_axis_name="c", subcore_axis_name="s")

    scratch = (
        (pltpu.VMEM @ vmesh)((per_w,), jnp.int32),
        (pltpu.VMEM @ vmesh)((per_w,), jnp.int32),
        pltpu.VMEM_SHARED((vocab,), jnp.int32),
        pltpu.SemaphoreType.DMA @ vmesh,
        pltpu.SemaphoreType.DMA @ vmesh,
        pltpu.SemaphoreType.DMA @ smesh,
        pltpu.SemaphoreType.REGULAR @ vmesh,
    )

    def scs_fn(ids, table, out, idx_v, vals_v, table_s, s_in, s_g, s_st, ready):
        del ids, out, idx_v, vals_v, s_in, s_g
        pltpu.async_copy(table, table_s, s_st).wait()
        for t in range(_NS):
            pltpu.semaphore_signal(ready, 1, device_id={"s": t})

    def tec_fn(ids, table, out, idx_v, vals_v, table_s, s_in, s_g, s_st, ready):
        del table, s_st
        sid = lax.axis_index("s")
        wid = sid * _NC + lax.axis_index("c")
        base = wid * per_w
        in_c = pltpu.async_copy(ids.at[pl.ds(base, per_w)], idx_v, s_in)
        in_c.wait()
        pl.semaphore_wait(ready, 1)
        pltpu.async_copy(table_s.at[idx_v], vals_v, s_g).wait()
        pltpu.sync_copy(vals_v, out.at[pl.ds(base, per_w)])

    return mpmd.mpmd_map(
        [(smesh, scs_fn), (vmesh, tec_fn)],
        out_types=[jax.ShapeDtypeStruct((n_tokens,), jnp.int32)],
        scratch_types=scratch,
    )


def kernel(input, hash):
    b, s = input.shape
    n = b * s
    ids = input.astype(jnp.int32).reshape(n)
    (out,) = _make_router(n, hash.shape[0])(ids, hash.astype(jnp.int32))
    return out.reshape(b, s).astype(hash.dtype)
